# baseline (speedup 1.0000x reference)
import functools

import jax
import jax.numpy as jnp
from jax import lax
from jax.experimental import pallas as pl
from jax.experimental.pallas import tpu as pltpu
from jax.experimental.pallas import tpu_sc as plsc

_NUM_SAMPLES = 4
_EPS = 1e-8
_NW = 32
_L = 16

_NOP = 20480
_NQP = 1536
_NSP = 6144
_NVP = 10240
_MB = 512
_NCH = 2048
_FWD_STEPS = _NQP // _MB
_REV_STEPS = _NSP // _MB
_STEPS = _FWD_STEPS + _REV_STEPS


def _sc_body(nv_real, ovf_hbm, svf_hbm, of_hbm, sf_hbm, cf_hbm, fp_hbm,
             ob_hbm, ovg_hbm, qb_hbm, sp_hbm, fps_hbm,
             ovv, svv, ofv, sfv, cfv, fpv, obv, ovgv, qbv, spv, fpsv):
    w = lax.axis_index("s") * 2 + lax.axis_index("c")

    pltpu.sync_copy(ovf_hbm, ovv)
    pltpu.sync_copy(svf_hbm, svv)
    pltpu.sync_copy(of_hbm.at[pl.ds(w * 1920, 1920)], ofv)
    pltpu.sync_copy(sf_hbm.at[pl.ds(w * 144, 144)], sfv)
    pltpu.sync_copy(cf_hbm.at[pl.ds(w * 576, 576)], cfv)
    pltpu.sync_copy(fp_hbm.at[pl.ds(w * 48, 48)], fpv)

    ones = jnp.ones((_L,), jnp.float32)
    lane = lax.iota(jnp.int32, _L)

    def ob_loop(g, _):
        s = pl.ds(g * _L, _L)
        l = g * _L + lane
        v0 = plsc.load_gather(ofv, [l * 3])
        v1 = plsc.load_gather(ofv, [l * 3 + 1])
        v2 = plsc.load_gather(ofv, [l * 3 + 2])
        coords = []
        for c in range(3):
            gc = (plsc.load_gather(ovv, [v0 * 3 + c])
                  + plsc.load_gather(ovv, [v1 * 3 + c])
                  + plsc.load_gather(ovv, [v2 * 3 + c]))
            coords.append(gc / 3.0)
        nrm = coords[0] * coords[0] + coords[1] * coords[1] + coords[2] * coords[2]
        for c in range(3):
            obv[pl.ds(c * 640 + g * _L, _L)] = coords[c]
        obv[pl.ds(3 * 640 + g * _L, _L)] = ones
        obv[pl.ds(4 * 640 + g * _L, _L)] = nrm
        return _

    def qb_loop(g, _):
        l = g * _L + lane
        v0 = plsc.load_gather(sfv, [l * 3])
        v1 = plsc.load_gather(sfv, [l * 3 + 1])
        v2 = plsc.load_gather(sfv, [l * 3 + 2])
        coords = []
        for c in range(3):
            gc = (plsc.load_gather(svv, [v0 * 3 + c])
                  + plsc.load_gather(svv, [v1 * 3 + c])
                  + plsc.load_gather(svv, [v2 * 3 + c]))
            coords.append(gc / 3.0)
        nrm = coords[0] * coords[0] + coords[1] * coords[1] + coords[2] * coords[2]
        vals = [-2.0 * coords[0], -2.0 * coords[1], -2.0 * coords[2], nrm, ones]
        for r in range(5):
            plsc.store_scatter(qbv, [l * 5 + r], vals[r])
        return _

    def sp_loop(g, _):
        s = pl.ds(g * _L, _L)
        l = g * _L + lane
        fl = lax.shift_right_logical(l, 2)
        i0 = plsc.load_gather(sfv, [fl * 3])
        i1 = plsc.load_gather(sfv, [fl * 3 + 1])
        i2 = plsc.load_gather(sfv, [fl * 3 + 2])
        ca = plsc.load_gather(cfv, [l * 3])
        cb = plsc.load_gather(cfv, [l * 3 + 1])
        cc = plsc.load_gather(cfv, [l * 3 + 2])
        coords = []
        for c in range(3):
            sc_ = (ca * plsc.load_gather(svv, [i0 * 3 + c])
                   + cb * plsc.load_gather(svv, [i1 * 3 + c])
                   + cc * plsc.load_gather(svv, [i2 * 3 + c]))
            coords.append(sc_)
        nrm = coords[0] * coords[0] + coords[1] * coords[1] + coords[2] * coords[2]
        vals = [-2.0 * coords[0], -2.0 * coords[1], -2.0 * coords[2], nrm, ones]
        for r in range(5):
            plsc.store_scatter(spv, [l * 5 + r], vals[r])
        fpsv[s] = plsc.load_gather(fpv, [fl])
        return _

    def ovg_loop(g, _):
        s = pl.ds(g * _L, _L)
        rows = jnp.minimum(w * (_NVP // _NW) + g * _L + lane, nv_real - 1)
        coords = [plsc.load_gather(ovv, [rows * 3 + c]) for c in range(3)]
        nrm = coords[0] * coords[0] + coords[1] * coords[1] + coords[2] * coords[2]
        for c in range(3):
            ovgv[pl.ds(c * 320 + g * _L, _L)] = coords[c]
        ovgv[pl.ds(3 * 320 + g * _L, _L)] = ones
        ovgv[pl.ds(4 * 320 + g * _L, _L)] = nrm
        return _

    lax.fori_loop(0, _NOP // _NW // _L, ob_loop, 0)
    lax.fori_loop(0, _NQP // _NW // _L, qb_loop, 0)
    lax.fori_loop(0, _NSP // _NW // _L, sp_loop, 0)
    lax.fori_loop(0, _NVP // _NW // _L, ovg_loop, 0)

    for c in range(5):
        pltpu.sync_copy(obv.at[pl.ds(c * 640, 640)],
                        ob_hbm.at[pl.ds(c * _NOP + w * 640, 640)])
        pltpu.sync_copy(ovgv.at[pl.ds(c * 320, 320)],
                        ovg_hbm.at[pl.ds(c * _NVP + w * 320, 320)])
    pltpu.sync_copy(qbv, qb_hbm.at[pl.ds(w * 240, 240)])
    pltpu.sync_copy(spv, sp_hbm.at[pl.ds(w * 960, 960)])
    pltpu.sync_copy(fpsv, fps_hbm.at[pl.ds(w * 192, 192)])


def _sc_stage(ovf, svf, off, sff, cff, fpp, nv_real, interpret=False):
    f32, i32 = jnp.float32, jnp.int32
    mesh = plsc.VectorSubcoreMesh(core_axis_name="c", subcore_axis_name="s")
    out_type = (
        jax.ShapeDtypeStruct((5 * _NOP,), f32),
        jax.ShapeDtypeStruct((5 * _NVP,), f32),
        jax.ShapeDtypeStruct((_NQP * 5,), f32),
        jax.ShapeDtypeStruct((_NSP * 5,), f32),
        jax.ShapeDtypeStruct((_NSP,), f32),
    )
    scratch = [
        pltpu.VMEM(ovf.shape, f32), pltpu.VMEM(svf.shape, f32),
        pltpu.VMEM((1920,), i32), pltpu.VMEM((144,), i32),
        pltpu.VMEM((576,), f32), pltpu.VMEM((48,), f32),
        pltpu.VMEM((5 * 640,), f32), pltpu.VMEM((5 * 320,), f32),
        pltpu.VMEM((240,), f32), pltpu.VMEM((960,), f32),
        pltpu.VMEM((192,), f32),
    ]
    fn = pl.kernel(functools.partial(_sc_body, nv_real), out_type,
                   mesh=mesh, scratch_types=scratch,
                   compiler_params=pltpu.CompilerParams(
                       needs_layout_passes=False),
                   interpret=interpret)
    return fn(ovf, svf, off, sff, cff, fpp)


def _tc_body(nreal_q, qr, ob, sr, ov, fpf, fps, out, acc, dacc):
    i = pl.program_id(0)

    @pl.when(i == 0)
    def _init():
        acc[0] = 0.0001 * (nreal_q - jnp.sum(fpf[...]))
        acc[1] = 0.0
        acc[2] = 0.0

    def min_block(q_ref, c_ref, qs, n_chunks):
        q3 = q_ref[pl.ds(qs, _MB), 0:3]

        for n in range(n_chunks):
            cross = lax.dot_general(
                q3, c_ref[0:3, pl.ds(n * _NCH, _NCH)],
                dimension_numbers=(((1,), (0,)), ((), ())),
                preferred_element_type=jnp.float32)
            sq = c_ref[4:5, pl.ds(n * _NCH, _NCH)] + cross
            dacc[...] = sq if n == 0 else jnp.minimum(dacc[...], sq)

        qn = q_ref[pl.ds(qs, _MB), 3]
        m = qn + jnp.min(dacc[...], axis=1)
        return jnp.sqrt(jnp.maximum(m, 0.0))

    @pl.when(i < _FWD_STEPS)
    def _fwd():
        qs = pl.multiple_of(i * _MB, _MB)
        dist = min_block(qr, ob, qs, _NOP // _NCH)
        acc[0] += jnp.sum(fpf[0, pl.ds(qs, _MB)] * dist)

    @pl.when(i >= _FWD_STEPS)
    def _rev():
        qs = pl.multiple_of((i - _FWD_STEPS) * _MB, _MB)
        dist = min_block(sr, ov, qs, _NVP // _NCH)
        acc[1] += jnp.sum(fps[0, pl.ds(qs, _MB)] * dist)
        acc[2] = jnp.maximum(acc[2], jnp.max(dist))

    @pl.when(i == _STEPS - 1)
    def _fin():
        out[0, 0] = acc[0] + acc[1] * 0.1 / (acc[2] + _EPS)


def _nn_loss(qr, ob, sr, ov, fpf, fps, nreal_q, interpret=False):
    full = lambda s: pl.BlockSpec(s, lambda i: (0,) * len(s))
    return pl.pallas_call(
        functools.partial(_tc_body, nreal_q),
        grid=(_STEPS,),
        in_specs=[full((_NQP, 5)), full((5, _NOP)), full((_NSP, 5)),
                  full((5, _NVP)), full((1, _NQP)), full((1, _NSP))],
        out_specs=pl.BlockSpec(memory_space=pltpu.SMEM),
        out_shape=jax.ShapeDtypeStruct((1, 1), jnp.float32),
        scratch_shapes=[pltpu.SMEM((3,), jnp.float32),
                        pltpu.VMEM((_MB, _NCH), jnp.float32)],
        interpret=interpret,
    )(qr, ob, sr, ov, fpf, fps)


def _edge_pad_rows(x, n_pad):
    return jnp.pad(x, ((0, n_pad - x.shape[0]), (0, 0)), mode="edge")


def kernel(original_vertices, original_faces, simplified_vertices,
           simplified_faces, face_probabilities, interpret=False):
    nf = simplified_faces.shape[0]
    nv = original_vertices.shape[0]
    fp = face_probabilities[:nf]

    skey = jax.random.key(42)
    ka, kb = jax.random.split(skey)
    sqrt_r1 = jnp.sqrt(jax.random.uniform(ka, (nf * _NUM_SAMPLES, 1),
                                          dtype=jnp.float32))
    r2 = jax.random.uniform(kb, (nf * _NUM_SAMPLES, 1), dtype=jnp.float32)
    coef = jnp.concatenate([1.0 - sqrt_r1, sqrt_r1 * (1.0 - r2),
                            sqrt_r1 * r2], axis=1)

    off = _edge_pad_rows(original_faces.astype(jnp.int32), _NOP).reshape(-1)
    sff = _edge_pad_rows(simplified_faces.astype(jnp.int32), _NQP).reshape(-1)
    cff = _edge_pad_rows(coef, _NSP).reshape(-1)
    fpp = jnp.pad(fp, (0, _NQP - nf))

    ob, ovg, qbf, spf, fps = _sc_stage(
        original_vertices.reshape(-1), simplified_vertices.reshape(-1),
        off, sff, cff, fpp, nv, interpret=interpret)

    out = _nn_loss(qbf.reshape(_NQP, 5), ob.reshape(5, _NOP),
                   spf.reshape(_NSP, 5), ovg.reshape(5, _NVP),
                   fpp[None, :], fps[None, :], float(nf),
                   interpret=interpret)
    return out.reshape(())


# speedup vs baseline: 1.1270x; 1.1270x over previous
import functools

import numpy as np

import jax
import jax.numpy as jnp
from jax import lax
from jax.experimental import pallas as pl
from jax.experimental.pallas import tpu as pltpu
from jax.experimental.pallas import tpu_sc as plsc

_NUM_SAMPLES = 4
_EPS = 1e-8
_NW = 32
_L = 16

_NOP = 20480
_NQP = 1536
_NSP = 6144
_NVP = 10240
_MB = 512
_NCH = 2048
_FWD_STEPS = _NQP // _MB
_REV_STEPS = _NSP // _MB
_STEPS = _FWD_STEPS + _REV_STEPS


def _sc_body(nv_real, ovf_hbm, svf_hbm, of_hbm, sf_hbm, cf_hbm, fp_hbm,
             ob_hbm, ovg_hbm, qb_hbm, sp_hbm, fps_hbm,
             ovv, svv, ofv, sfv, cfv, fpv, obv, ovgv, qbv, spv, fpsv):
    w = lax.axis_index("s") * 2 + lax.axis_index("c")

    pltpu.sync_copy(ovf_hbm, ovv)
    pltpu.sync_copy(svf_hbm, svv)
    pltpu.sync_copy(of_hbm.at[pl.ds(w * 1920, 1920)], ofv)
    pltpu.sync_copy(sf_hbm.at[pl.ds(w * 144, 144)], sfv)
    pltpu.sync_copy(cf_hbm.at[pl.ds(w * 576, 576)], cfv)
    pltpu.sync_copy(fp_hbm.at[pl.ds(w * 48, 48)], fpv)

    ones = jnp.ones((_L,), jnp.float32)
    lane = lax.iota(jnp.int32, _L)

    def ob_loop(g, _):
        s = pl.ds(g * _L, _L)
        l = g * _L + lane
        v0 = plsc.load_gather(ofv, [l * 3])
        v1 = plsc.load_gather(ofv, [l * 3 + 1])
        v2 = plsc.load_gather(ofv, [l * 3 + 2])
        coords = []
        for c in range(3):
            gc = (plsc.load_gather(ovv, [v0 * 3 + c])
                  + plsc.load_gather(ovv, [v1 * 3 + c])
                  + plsc.load_gather(ovv, [v2 * 3 + c]))
            coords.append(gc / 3.0)
        nrm = coords[0] * coords[0] + coords[1] * coords[1] + coords[2] * coords[2]
        for c in range(3):
            obv[c, s] = coords[c]
        obv[3, s] = ones
        obv[4, s] = nrm
        return _

    def qb_loop(g, _):
        l = g * _L + lane
        v0 = plsc.load_gather(sfv, [l * 3])
        v1 = plsc.load_gather(sfv, [l * 3 + 1])
        v2 = plsc.load_gather(sfv, [l * 3 + 2])
        coords = []
        for c in range(3):
            gc = (plsc.load_gather(svv, [v0 * 3 + c])
                  + plsc.load_gather(svv, [v1 * 3 + c])
                  + plsc.load_gather(svv, [v2 * 3 + c]))
            coords.append(gc / 3.0)
        nrm = coords[0] * coords[0] + coords[1] * coords[1] + coords[2] * coords[2]
        vals = [-2.0 * coords[0], -2.0 * coords[1], -2.0 * coords[2], nrm, ones]
        for r in range(5):
            plsc.store_scatter(qbv, [l * 5 + r], vals[r])
        return _

    def sp_loop(g, _):
        s = pl.ds(g * _L, _L)
        l = g * _L + lane
        fl = lax.shift_right_logical(l, 2)
        i0 = plsc.load_gather(sfv, [fl * 3])
        i1 = plsc.load_gather(sfv, [fl * 3 + 1])
        i2 = plsc.load_gather(sfv, [fl * 3 + 2])
        ca = plsc.load_gather(cfv, [l * 3])
        cb = plsc.load_gather(cfv, [l * 3 + 1])
        cc = plsc.load_gather(cfv, [l * 3 + 2])
        coords = []
        for c in range(3):
            sc_ = (ca * plsc.load_gather(svv, [i0 * 3 + c])
                   + cb * plsc.load_gather(svv, [i1 * 3 + c])
                   + cc * plsc.load_gather(svv, [i2 * 3 + c]))
            coords.append(sc_)
        nrm = coords[0] * coords[0] + coords[1] * coords[1] + coords[2] * coords[2]
        vals = [-2.0 * coords[0], -2.0 * coords[1], -2.0 * coords[2], nrm, ones]
        for r in range(5):
            plsc.store_scatter(spv, [l * 5 + r], vals[r])
        fpsv[s] = plsc.load_gather(fpv, [fl])
        return _

    def ovg_loop(g, _):
        s = pl.ds(g * _L, _L)
        rows = jnp.minimum(w * (_NVP // _NW) + g * _L + lane, nv_real - 1)
        coords = [plsc.load_gather(ovv, [rows * 3 + c]) for c in range(3)]
        nrm = coords[0] * coords[0] + coords[1] * coords[1] + coords[2] * coords[2]
        for c in range(3):
            ovgv[c, s] = coords[c]
        ovgv[3, s] = ones
        ovgv[4, s] = nrm
        return _

    lax.fori_loop(0, _NOP // _NW // _L, ob_loop, 0)
    lax.fori_loop(0, _NQP // _NW // _L, qb_loop, 0)
    lax.fori_loop(0, _NSP // _NW // _L, sp_loop, 0)
    lax.fori_loop(0, _NVP // _NW // _L, ovg_loop, 0)

    pltpu.sync_copy(obv, ob_hbm.at[w])
    pltpu.sync_copy(ovgv, ovg_hbm.at[w])
    pltpu.sync_copy(qbv, qb_hbm.at[pl.ds(w * 240, 240)])
    pltpu.sync_copy(spv, sp_hbm.at[pl.ds(w * 960, 960)])
    pltpu.sync_copy(fpsv, fps_hbm.at[pl.ds(w * 192, 192)])


def _sc_stage(ovf, svf, off, sff, cff, fpp, nv_real, interpret=False):
    f32, i32 = jnp.float32, jnp.int32
    mesh = plsc.VectorSubcoreMesh(core_axis_name="c", subcore_axis_name="s")
    out_type = (
        jax.ShapeDtypeStruct((_NW, 5, _NOP // _NW), f32),
        jax.ShapeDtypeStruct((_NW, 5, _NVP // _NW), f32),
        jax.ShapeDtypeStruct((_NQP * 5,), f32),
        jax.ShapeDtypeStruct((_NSP * 5,), f32),
        jax.ShapeDtypeStruct((_NSP,), f32),
    )
    scratch = [
        pltpu.VMEM(ovf.shape, f32), pltpu.VMEM(svf.shape, f32),
        pltpu.VMEM((1920,), i32), pltpu.VMEM((144,), i32),
        pltpu.VMEM((576,), f32), pltpu.VMEM((48,), f32),
        pltpu.VMEM((5, 640), f32), pltpu.VMEM((5, 320), f32),
        pltpu.VMEM((240,), f32), pltpu.VMEM((960,), f32),
        pltpu.VMEM((192,), f32),
    ]
    fn = pl.kernel(functools.partial(_sc_body, nv_real), out_type,
                   mesh=mesh, scratch_types=scratch,
                   compiler_params=pltpu.CompilerParams(
                       needs_layout_passes=False),
                   interpret=interpret)
    return fn(ovf, svf, off, sff, cff, fpp)


def _tc_body(nreal_q, qr, ob, sr, ov, fpf, fps, out, acc, dacc):
    i = pl.program_id(0)

    @pl.when(i == 0)
    def _init():
        acc[0] = 0.0001 * (nreal_q - jnp.sum(fpf[...]))
        acc[1] = 0.0
        acc[2] = 0.0

    def min_block(q_ref, c_ref, qs, n_chunks):
        q3 = q_ref[pl.ds(qs, _MB), 0:3]

        for n in range(n_chunks):
            cross = lax.dot_general(
                q3, c_ref[0:3, pl.ds(n * _NCH, _NCH)],
                dimension_numbers=(((1,), (0,)), ((), ())),
                preferred_element_type=jnp.float32)
            sq = c_ref[4:5, pl.ds(n * _NCH, _NCH)] + cross
            dacc[...] = sq if n == 0 else jnp.minimum(dacc[...], sq)

        qn = q_ref[pl.ds(qs, _MB), 3]
        m = qn + jnp.min(dacc[...], axis=1)
        return jnp.sqrt(jnp.maximum(m, 0.0))

    @pl.when(i < _FWD_STEPS)
    def _fwd():
        qs = pl.multiple_of(i * _MB, _MB)
        dist = min_block(qr, ob, qs, _NOP // _NCH)
        acc[0] += jnp.sum(fpf[0, pl.ds(qs, _MB)] * dist)

    @pl.when(i >= _FWD_STEPS)
    def _rev():
        qs = pl.multiple_of((i - _FWD_STEPS) * _MB, _MB)
        dist = min_block(sr, ov, qs, _NVP // _NCH)
        acc[1] += jnp.sum(fps[0, pl.ds(qs, _MB)] * dist)
        acc[2] = jnp.maximum(acc[2], jnp.max(dist))

    @pl.when(i == _STEPS - 1)
    def _fin():
        out[0, 0] = acc[0] + acc[1] * 0.1 / (acc[2] + _EPS)


def _nn_loss(qr, ob, sr, ov, fpf, fps, nreal_q, interpret=False):
    full = lambda s: pl.BlockSpec(s, lambda i: (0,) * len(s))
    return pl.pallas_call(
        functools.partial(_tc_body, nreal_q),
        grid=(_STEPS,),
        in_specs=[full((_NQP, 5)), full((5, _NOP)), full((_NSP, 5)),
                  full((5, _NVP)), full((1, _NQP)), full((1, _NSP))],
        out_specs=pl.BlockSpec(memory_space=pltpu.SMEM),
        out_shape=jax.ShapeDtypeStruct((1, 1), jnp.float32),
        scratch_shapes=[pltpu.SMEM((3,), jnp.float32),
                        pltpu.VMEM((_MB, _NCH), jnp.float32)],
        interpret=interpret,
    )(qr, ob, sr, ov, fpf, fps)


def _edge_pad_rows(x, n_pad):
    return jnp.pad(x, ((0, n_pad - x.shape[0]), (0, 0)), mode="edge")


@functools.lru_cache(maxsize=None)
def _coef_flat(nf):
    with jax.ensure_compile_time_eval(), \
            jax.default_device(jax.devices("cpu")[0]):
        ka, kb = jax.random.split(jax.random.key(42))
        sqrt_r1 = jnp.sqrt(jax.random.uniform(ka, (nf * _NUM_SAMPLES, 1),
                                              dtype=jnp.float32))
        r2 = jax.random.uniform(kb, (nf * _NUM_SAMPLES, 1), dtype=jnp.float32)
        coef = jnp.concatenate([1.0 - sqrt_r1, sqrt_r1 * (1.0 - r2),
                                sqrt_r1 * r2], axis=1)
        coef = jnp.pad(coef, ((0, _NSP - nf * _NUM_SAMPLES), (0, 0)),
                       mode="edge").reshape(-1)
        return np.asarray(coef)


def kernel(original_vertices, original_faces, simplified_vertices,
           simplified_faces, face_probabilities, interpret=False):
    nf = simplified_faces.shape[0]
    nv = original_vertices.shape[0]
    fp = face_probabilities[:nf]

    off = _edge_pad_rows(original_faces.astype(jnp.int32), _NOP).reshape(-1)
    sff = _edge_pad_rows(simplified_faces.astype(jnp.int32), _NQP).reshape(-1)
    cff = jnp.asarray(_coef_flat(nf))
    fpp = jnp.pad(fp, (0, _NQP - nf))

    ob, ovg, qbf, spf, fps = _sc_stage(
        original_vertices.reshape(-1), simplified_vertices.reshape(-1),
        off, sff, cff, fpp, nv, interpret=interpret)

    out = _nn_loss(qbf.reshape(_NQP, 5), ob.transpose(1, 0, 2).reshape(5, _NOP),
                   spf.reshape(_NSP, 5), ovg.transpose(1, 0, 2).reshape(5, _NVP),
                   fpp[None, :], fps[None, :], float(nf),
                   interpret=interpret)
    return out.reshape(())
